# baseline (speedup 1.0000x reference)
import functools

import jax
import jax.numpy as jnp
from jax import lax
from jax.experimental import pallas as pl
from jax.experimental.pallas import tpu as pltpu
from jax.experimental.pallas import tpu_sc as plsc

_E = 320000
_N = 10000
_D = 128
_NB = _D // 16
_NC = 2
_NS = 16
_NW = _NC * _NS
_PER_W = _E // _NW
_C = 80
_NCH = _PER_W // _C
_G = _C // 16
_STG = _N // _NS


@functools.partial(
    pl.kernel,
    mesh=plsc.VectorSubcoreMesh(core_axis_name="c", subcore_axis_name="s"),
    out_type=jax.ShapeDtypeStruct((_E,), jnp.float32),
    compiler_params=pltpu.CompilerParams(
        needs_layout_passes=False, use_tc_tiling_on_sc=False),
    scratch_types=[
        pltpu.VMEM_SHARED((_N, _D // 2), jnp.int32),
        pltpu.VMEM((_C,), jnp.int32),
        pltpu.VMEM((_C,), jnp.int32),
        pltpu.VMEM((_C,), jnp.int32),
        pltpu.VMEM((_C,), jnp.int32),
        pltpu.VMEM((_C, _D // 2), jnp.int32),
        pltpu.VMEM((_C, _D // 2), jnp.int32),
        pltpu.VMEM((_C, _D // 2), jnp.int32),
        pltpu.VMEM((_C, _D // 2), jnp.int32),
        pltpu.VMEM((_C,), jnp.float32),
        pltpu.VMEM((_C,), jnp.float32),
        pltpu.VMEM((_G * 272,), jnp.float32),
        pltpu.SemaphoreType.DMA,
        pltpu.SemaphoreType.DMA,
        pltpu.SemaphoreType.DMA,
        pltpu.SemaphoreType.DMA,
        pltpu.SemaphoreType.DMA,
        pltpu.SemaphoreType.DMA,
        pltpu.SemaphoreType.DMA,
        pltpu.SemaphoreType.DMA,
        pltpu.SemaphoreType.DMA,
        pltpu.SemaphoreType.DMA,
    ],
)
def _edge_dot(table, src, dst, out, shtab, ia0, ib0, ia1, ib1, ra0,
              rb0, ra1, rb1, ov0, ov1, tsc, sia0, sib0, sia1, sib1, sa0, sb0,
              sa1, sb1, so0, so1):
    wid = lax.axis_index("s") * _NC + lax.axis_index("c")
    base = pl.multiple_of(wid * _PER_W, 8)
    sid = lax.axis_index("s")

    roff = pl.multiple_of(sid * _STG, 8)
    pltpu.sync_copy(table.at[pl.ds(roff, _STG)], shtab.at[pl.ds(roff, _STG)])
    plsc.subcore_barrier()

    iota = lax.iota(jnp.int32, 16)
    p17 = iota * 17

    def copy_idx(ch, ia, ib, sia, sib):
        off = pl.multiple_of(base + ch * _C, 8)
        pltpu.async_copy(src.at[pl.ds(off, _C)], ia, sia)
        pltpu.async_copy(dst.at[pl.ds(off, _C)], ib, sib)

    def wait_idx(ia, ib, sia, sib):
        pltpu.make_async_copy(src.at[pl.ds(0, _C)], ia, sia).wait()
        pltpu.make_async_copy(dst.at[pl.ds(0, _C)], ib, sib).wait()

    def issue_rows(ia, ib, ra, rb, sa, sb):
        pltpu.async_copy(shtab.at[ia], ra, sa)
        pltpu.async_copy(shtab.at[ib], rb, sb)

    def drain_rows(ia, ib, ra, rb, sa, sb):
        pltpu.make_async_copy(shtab.at[ia], ra, sa).wait()
        pltpu.make_async_copy(shtab.at[ib], rb, sb).wait()

    def compute(ch, rows_a, rows_b, ov, so):
        @plsc.parallel_loop(0, _C, unroll=10)
        def _edge(e):
            prods = []
            for k in range(_D // 32):
                pa = plsc.bitcast(rows_a[e, pl.ds(k * 16, 16)], jnp.bfloat16)
                pb = plsc.bitcast(rows_b[e, pl.ds(k * 16, 16)], jnp.bfloat16)
                prods.append(pa * pb)
            a0, a1 = plsc.unpack(prods[0] + prods[1],
                                 format=plsc.PackFormat.INTERLEAVED)
            b0, b1 = plsc.unpack(prods[2] + prods[3],
                                 format=plsc.PackFormat.INTERLEAVED)
            s = (a0 + a1) + (b0 + b1)
            plsc.store_scatter(tsc, [iota + e * 17], s)

        @pl.when(ch >= 2)
        def _():
            pltpu.make_async_copy(ov, out.at[pl.ds(0, _C)], so).wait()

        @plsc.parallel_loop(0, _G, unroll=_G)
        def _grp(g):
            pbase = p17 + g * 272
            acc = plsc.load_gather(tsc, [pbase])
            for k in range(1, 16):
                acc = acc + plsc.load_gather(tsc, [pbase + k])
            acc = 1.0 / (1.0 + jnp.exp(-acc))
            ov[pl.ds(g * 16, 16)] = acc

        pltpu.async_copy(ov, out.at[pl.ds(base + ch * _C, _C)], so)

    copy_idx(0, ia0, ib0, sia0, sib0)
    copy_idx(1, ia1, ib1, sia1, sib1)
    wait_idx(ia0, ib0, sia0, sib0)
    issue_rows(ia0, ib0, ra0, rb0, sa0, sb0)
    wait_idx(ia1, ib1, sia1, sib1)
    issue_rows(ia1, ib1, ra1, rb1, sa1, sb1)

    def body2(i, carry):
        c0 = i * 2

        drain_rows(ia0, ib0, ra0, rb0, sa0, sb0)
        copy_idx(c0 + 2, ia0, ib0, sia0, sib0)
        compute(c0, ra0, rb0, ov0, so0)
        wait_idx(ia0, ib0, sia0, sib0)
        issue_rows(ia0, ib0, ra0, rb0, sa0, sb0)

        drain_rows(ia1, ib1, ra1, rb1, sa1, sb1)

        @pl.when(c0 + 3 < _NCH)
        def _():
            copy_idx(c0 + 3, ia1, ib1, sia1, sib1)

        compute(c0 + 1, ra1, rb1, ov1, so1)

        @pl.when(c0 + 3 < _NCH)
        def _():
            wait_idx(ia1, ib1, sia1, sib1)
            issue_rows(ia1, ib1, ra1, rb1, sa1, sb1)

        return carry

    lax.fori_loop(0, (_NCH - 1) // 2, body2, 0)

    drain_rows(ia0, ib0, ra0, rb0, sa0, sb0)
    compute(_NCH - 1, ra0, rb0, ov0, so0)
    pltpu.make_async_copy(ov0, out.at[pl.ds(0, _C)], so0).wait()
    pltpu.make_async_copy(ov1, out.at[pl.ds(0, _C)], so1).wait()


@jax.jit
def kernel(z, pairs):
    tab = jax.lax.bitcast_convert_type(
        z[2].astype(jnp.bfloat16).reshape(_N, _D // 2, 2), jnp.int32)
    return _edge_dot(tab, pairs[0], pairs[1])


# speedup vs baseline: 1.1716x; 1.1716x over previous
import functools

import jax
import jax.numpy as jnp
from jax import lax
from jax.experimental import pallas as pl
from jax.experimental.pallas import tpu as pltpu
from jax.experimental.pallas import tpu_sc as plsc

_E = 320000
_N = 10000
_D = 128
_NC = 2
_NS = 16
_NW = _NC * _NS
_PER_W = _E // _NW
_C = 80
_NCH = _PER_W // _C
_G = _C // 16


@functools.partial(
    pl.kernel,
    mesh=plsc.VectorSubcoreMesh(core_axis_name="c", subcore_axis_name="s"),
    out_type=jax.ShapeDtypeStruct((_E,), jnp.float32),
    compiler_params=pltpu.CompilerParams(
        needs_layout_passes=False, use_tc_tiling_on_sc=False),
    scratch_types=[
        pltpu.VMEM_SHARED((_N, _D // 2), jnp.int32),
        pltpu.VMEM((_C, _D), jnp.float32),
        pltpu.VMEM((_C, _D // 2), jnp.int32),
        pltpu.VMEM((_PER_W,), jnp.int32),
        pltpu.VMEM((_PER_W,), jnp.int32),
        pltpu.VMEM((_C, _D // 2), jnp.int32),
        pltpu.VMEM((_C, _D // 2), jnp.int32),
        pltpu.VMEM((_C, _D // 2), jnp.int32),
        pltpu.VMEM((_C, _D // 2), jnp.int32),
        pltpu.VMEM((_C,), jnp.float32),
        pltpu.VMEM((_C,), jnp.float32),
        pltpu.VMEM((_G * 272,), jnp.float32),
        pltpu.SemaphoreType.DMA,
        pltpu.SemaphoreType.DMA,
        pltpu.SemaphoreType.DMA,
        pltpu.SemaphoreType.DMA,
        pltpu.SemaphoreType.DMA,
        pltpu.SemaphoreType.DMA,
        pltpu.SemaphoreType.DMA,
        pltpu.SemaphoreType.DMA,
    ],
)
def _edge_dot(table, src, dst, out, shtab, raf, bb, isrc, idst, ra0,
              rb0, ra1, rb1, ov0, ov1, tsc, sis, sid_, sa0, sb0,
              sa1, sb1, so0, so1):
    wid = lax.axis_index("s") * _NC + lax.axis_index("c")
    base = pl.multiple_of(wid * _PER_W, 8)
    sid = lax.axis_index("s")

    pltpu.async_copy(src.at[pl.ds(base, _PER_W)], isrc, sis)
    pltpu.async_copy(dst.at[pl.ds(base, _PER_W)], idst, sid_)

    for t in range(8):
        c = sid + t * _NS

        @pl.when(c < _N // 80)
        def _():
            roff = pl.multiple_of(c * 80, 8)
            pltpu.sync_copy(table.at[pl.ds(roff, 80)], raf)

            @plsc.parallel_loop(0, _C, unroll=4)
            def _cv(r):
                for k in range(_D // 32):
                    x0 = raf[r, pl.ds(k * 32, 16)]
                    x1 = raf[r, pl.ds(k * 32 + 16, 16)]
                    bb[r, pl.ds(k * 16, 16)] = plsc.bitcast(
                        plsc.pack(x0, x1, format=plsc.PackFormat.INTERLEAVED),
                        jnp.int32)

            pltpu.sync_copy(bb, shtab.at[pl.ds(roff, 80)])

    plsc.subcore_barrier()
    pltpu.make_async_copy(src.at[pl.ds(0, _PER_W)], isrc, sis).wait()
    pltpu.make_async_copy(dst.at[pl.ds(0, _PER_W)], idst, sid_).wait()

    iota = lax.iota(jnp.int32, 16)
    p17 = iota * 17

    def issue_rows(ch, ra, rb, sa, sb):
        off = pl.multiple_of(ch * _C, 16)
        pltpu.async_copy(shtab.at[isrc.at[pl.ds(off, _C)]], ra, sa)
        pltpu.async_copy(shtab.at[idst.at[pl.ds(off, _C)]], rb, sb)

    def drain_rows(ra, rb, sa, sb):
        pltpu.make_async_copy(shtab.at[isrc.at[pl.ds(0, _C)]], ra, sa).wait()
        pltpu.make_async_copy(shtab.at[idst.at[pl.ds(0, _C)]], rb, sb).wait()

    def compute(ch, rows_a, rows_b, ov, so):
        @plsc.parallel_loop(0, _C, unroll=10)
        def _edge(e):
            prods = []
            for k in range(_D // 32):
                pa = plsc.bitcast(rows_a[e, pl.ds(k * 16, 16)], jnp.bfloat16)
                pb = plsc.bitcast(rows_b[e, pl.ds(k * 16, 16)], jnp.bfloat16)
                prods.append(pa * pb)
            a0, a1 = plsc.unpack(prods[0] + prods[1],
                                 format=plsc.PackFormat.INTERLEAVED)
            b0, b1 = plsc.unpack(prods[2] + prods[3],
                                 format=plsc.PackFormat.INTERLEAVED)
            s = (a0 + a1) + (b0 + b1)
            plsc.store_scatter(tsc, [iota + e * 17], s)

        @pl.when(ch >= 2)
        def _():
            pltpu.make_async_copy(ov, out.at[pl.ds(0, _C)], so).wait()

        @plsc.parallel_loop(0, _G, unroll=_G)
        def _grp(g):
            pbase = p17 + g * 272
            acc = plsc.load_gather(tsc, [pbase])
            for k in range(1, 16):
                acc = acc + plsc.load_gather(tsc, [pbase + k])
            acc = 1.0 / (1.0 + jnp.exp(-acc))
            ov[pl.ds(g * 16, 16)] = acc

        pltpu.async_copy(ov, out.at[pl.ds(base + ch * _C, _C)], so)

    issue_rows(0, ra0, rb0, sa0, sb0)
    issue_rows(1, ra1, rb1, sa1, sb1)

    def body2(i, carry):
        c0 = i * 2

        drain_rows(ra0, rb0, sa0, sb0)
        compute(c0, ra0, rb0, ov0, so0)
        issue_rows(c0 + 2, ra0, rb0, sa0, sb0)

        drain_rows(ra1, rb1, sa1, sb1)
        compute(c0 + 1, ra1, rb1, ov1, so1)

        @pl.when(c0 + 3 < _NCH)
        def _():
            issue_rows(c0 + 3, ra1, rb1, sa1, sb1)

        return carry

    lax.fori_loop(0, (_NCH - 1) // 2, body2, 0)

    drain_rows(ra0, rb0, sa0, sb0)
    compute(_NCH - 1, ra0, rb0, ov0, so0)
    pltpu.make_async_copy(ov0, out.at[pl.ds(0, _C)], so0).wait()
    pltpu.make_async_copy(ov1, out.at[pl.ds(0, _C)], so1).wait()


@jax.jit
def kernel(z, pairs):
    table = z[2]
    return _edge_dot(table, pairs[0], pairs[1])
